# baseline (device time: 28931 ns/iter reference)
import jax
import jax.numpy as jnp
from jax import lax
from jax.experimental import pallas as pl
from jax.experimental.pallas import tpu as pltpu

NZ = 4
T = 512
D = 512
F = 1024
EL = 2
NE = NZ * EL
PE = 80
P = EL * PE
S = NE * PE

ORDER = {0: (1, 2, 3), 1: (0, 2, 3), 2: (1, 3, 0), 3: (2, 1, 0)}


def kernel(x, assign, W1, W2):
    a2 = assign.reshape(T, 1)

    def body(x_ref, a_ref, w1_ref, w2_ref, out_ref,
             sxs, own, xg, yb, opad,
             sendx, recvx, sendp, recvp):
        my_x = lax.axis_index("x")
        my_y = lax.axis_index("y")
        my_z = lax.axis_index("z")

        def peer(j):
            return (my_x, my_y, j)

        barrier = pltpu.get_barrier_semaphore()
        for j in range(NZ):
            @pl.when(my_z != j)
            def _(j=j):
                pl.semaphore_signal(
                    barrier, inc=1,
                    device_id=peer(j), device_id_type=pl.DeviceIdType.MESH,
                )
        pl.semaphore_wait(barrier, NZ - 1)

        ecol = a_ref[...]
        onehot = (
            ecol == lax.broadcasted_iota(jnp.int32, (T, NE), 1)
        ).astype(jnp.float32)
        ltri = (
            lax.broadcasted_iota(jnp.int32, (T, T), 0)
            >= lax.broadcasted_iota(jnp.int32, (T, T), 1)
        ).astype(jnp.float32)
        incl = jnp.dot(ltri, onehot, preferred_element_type=jnp.float32)
        rank = jnp.sum(incl * onehot, axis=1, keepdims=True) - 1.0
        pos = (ecol.astype(jnp.float32) * PE + rank).astype(jnp.int32)

        def pm_for(j):
            return (
                lax.broadcasted_iota(jnp.int32, (T, P), 1) == pos - j * P
            ).astype(jnp.float32)

        def pack(pm):
            return lax.dot_general(
                pm, x_ref[...], (((0,), (0,)), ((), ())),
                preferred_element_type=jnp.float32,
            ).astype(jnp.bfloat16)

        def expert_rows(xs, le):
            h = jnp.maximum(
                jnp.dot(xs, w1_ref[le], preferred_element_type=jnp.float32),
                0.0,
            )
            return jnp.dot(h, w2_ref[le], preferred_element_type=jnp.float32)

        for me in range(NZ):
            @pl.when(my_z == me)
            def _(me=me):
                order = ORDER[me]

                for j in order:
                    sxs[j] = pack(pm_for(j))
                    pltpu.make_async_remote_copy(
                        src_ref=sxs.at[j], dst_ref=xg.at[me],
                        send_sem=sendx.at[j], recv_sem=recvx.at[me],
                        device_id=peer(j),
                        device_id_type=pl.DeviceIdType.MESH,
                    ).start()

                pm_own = pm_for(me)
                own[...] = pack(pm_own)
                for le in range(EL):
                    opad[pl.ds(me * P + le * PE, PE), :] = expert_rows(
                        own[pl.ds(le * PE, PE), :], le
                    ).astype(jnp.bfloat16)

                for j in order:
                    pltpu.make_async_remote_copy(
                        src_ref=sxs.at[j], dst_ref=xg.at[j],
                        send_sem=sendx.at[j], recv_sem=recvx.at[j],
                        device_id=peer(j),
                        device_id_type=pl.DeviceIdType.MESH,
                    ).wait_recv()
                    for le in range(EL):
                        yb[j, pl.ds(le * PE, PE), :] = expert_rows(
                            xg[j, pl.ds(le * PE, PE), :], le
                        ).astype(jnp.bfloat16)
                        pltpu.make_async_remote_copy(
                            src_ref=yb.at[j, pl.ds(le * PE, PE)],
                            dst_ref=opad.at[pl.ds(me * P + le * PE, PE)],
                            send_sem=sendp.at[j, le],
                            recv_sem=recvp.at[me, le],
                            device_id=peer(j),
                            device_id_type=pl.DeviceIdType.MESH,
                        ).start()

                out_ref[...] = jnp.dot(
                    pm_own.astype(jnp.bfloat16),
                    opad[pl.ds(me * P, P), :],
                    preferred_element_type=jnp.float32,
                )
                for j in order:
                    for le in range(EL):
                        pltpu.make_async_remote_copy(
                            src_ref=yb.at[j, pl.ds(le * PE, PE)],
                            dst_ref=opad.at[pl.ds(j * P + le * PE, PE)],
                            send_sem=sendp.at[j, le],
                            recv_sem=recvp.at[j, le],
                            device_id=peer(j),
                            device_id_type=pl.DeviceIdType.MESH,
                        ).wait_recv()
                    out_ref[...] += jnp.dot(
                        pm_for(j).astype(jnp.bfloat16),
                        opad[pl.ds(j * P, P), :],
                        preferred_element_type=jnp.float32,
                    )

                for j in order:
                    pltpu.make_async_remote_copy(
                        src_ref=sxs.at[j], dst_ref=xg.at[j],
                        send_sem=sendx.at[j], recv_sem=recvx.at[j],
                        device_id=peer(j),
                        device_id_type=pl.DeviceIdType.MESH,
                    ).wait_send()
                    for le in range(EL):
                        pltpu.make_async_remote_copy(
                            src_ref=yb.at[j, pl.ds(le * PE, PE)],
                            dst_ref=opad.at[pl.ds(j * P + le * PE, PE)],
                            send_sem=sendp.at[j, le],
                            recv_sem=recvp.at[j, le],
                            device_id=peer(j),
                            device_id_type=pl.DeviceIdType.MESH,
                        ).wait_send()

    return pl.pallas_call(
        body,
        out_shape=jax.ShapeDtypeStruct((T, D), jnp.float32),
        in_specs=[pl.BlockSpec(memory_space=pltpu.VMEM)] * 4,
        out_specs=pl.BlockSpec(memory_space=pltpu.VMEM),
        scratch_shapes=[
            pltpu.VMEM((NZ, P, D), jnp.bfloat16),
            pltpu.VMEM((P, D), jnp.bfloat16),
            pltpu.VMEM((NZ, P, D), jnp.bfloat16),
            pltpu.VMEM((NZ, P, D), jnp.bfloat16),
            pltpu.VMEM((S, D), jnp.bfloat16),
            pltpu.SemaphoreType.DMA((NZ,)),
            pltpu.SemaphoreType.DMA((NZ,)),
            pltpu.SemaphoreType.DMA((NZ, EL)),
            pltpu.SemaphoreType.DMA((NZ, EL)),
        ],
        compiler_params=pltpu.CompilerParams(collective_id=0),
    )(x, a2, W1, W2)
